# baseline (speedup 1.0000x reference)
import functools
import math

import jax
import jax.numpy as jnp
from jax import lax
from jax.experimental import pallas as pl
from jax.experimental.pallas import tpu as pltpu
from jax.experimental.pallas import tpu_sc as plsc

B, H, S, D = 1, 16, 2048, 128
N_ELEM = B * H * S * D
N_OUT = int(0.005 * N_ELEM)
SINK_LENGTH = 4
QMAX = 127.0
BLK_Q = 1024


_SC_NC, _SC_NS, _SC_L = 2, 16, 16
_PER_W = N_ELEM // _SC_NS
_CHUNK = 16384
_UNROLL = 8
_NB = 4096

_sc_mesh = plsc.VectorSubcoreMesh(core_axis_name="c", subcore_axis_name="s")


@functools.partial(
    pl.kernel,
    mesh=_sc_mesh,
    out_type=jax.ShapeDtypeStruct((16,), jnp.int32),
    scratch_types=[
        pltpu.VMEM((_CHUNK,), jnp.int32),
        pltpu.VMEM((_CHUNK,), jnp.int32),
        pltpu.VMEM((_SC_L, _NB), jnp.int32),
        pltpu.VMEM((1, _NB), jnp.int32),
        pltpu.VMEM((_NB,), jnp.int32),
        pltpu.VMEM((16,), jnp.int32),
        pltpu.VMEM_SHARED((_SC_NS, _NB), jnp.int32),
        pltpu.SemaphoreType.DMA,
        pltpu.SemaphoreType.DMA,
    ],
    compiler_params=pltpu.CompilerParams(needs_layout_passes=False),
)
def _sc_thresh(k_hbm, v_hbm, out_hbm,
               buf_a, buf_b, hist_v, res_v, scan_v, tb_v, sh_hist,
               sem_a, sem_b):
    core = lax.axis_index("c")
    sid = lax.axis_index("s")
    base = sid * _PER_W
    lanes = lax.iota(jnp.int32, _SC_L)
    ones = jnp.ones((_SC_L,), jnp.int32)
    zeros16 = jnp.zeros((_SC_L,), jnp.int32)
    signmask = jnp.full((_SC_L,), 0x7FFFFFFF, jnp.int32)
    iota16 = lax.iota(jnp.int32, 16)

    def _hist_pass(data_hbm, fs, fv, bs, bm):
        fs = jnp.full((_SC_L,), fs, jnp.int32)
        fv = jnp.full((_SC_L,), fv, jnp.int32)
        bs = jnp.full((_SC_L,), bs, jnp.int32)
        bm = jnp.full((_SC_L,), bm, jnp.int32)
        for r in range(_SC_L):
            @plsc.parallel_loop(0, _NB // 16, unroll=8)
            def _zero(j, r=r):
                hist_v[r, pl.ds(j * 16, 16)] = zeros16

        n_chunks = _PER_W // _CHUNK
        bufs = (buf_a, buf_b)
        sems = (sem_a, sem_b)
        handles = [pltpu.async_copy(
            data_hbm.at[pl.ds(base, _CHUNK)], buf_a, sem_a)]
        for c in range(n_chunks):
            if c + 1 < n_chunks:
                handles.append(pltpu.async_copy(
                    data_hbm.at[pl.ds(base + (c + 1) * _CHUNK, _CHUNK)],
                    bufs[(c + 1) % 2], sems[(c + 1) % 2]))
            handles[c].wait()
            buf_v = bufs[c % 2]

            @plsc.parallel_loop(0, _CHUNK // _SC_L, unroll=_UNROLL)
            def _vec(i, buf_v=buf_v):
                key = buf_v[pl.ds(i * _SC_L, _SC_L)] & signmask
                keep = lax.shift_right_logical(key, fs) == fv
                bucket = lax.shift_right_logical(key, bs) & bm
                plsc.addupdate_scatter(hist_v, [lanes, bucket], ones, mask=keep)

        @plsc.parallel_loop(0, _NB // 16, unroll=2)
        def _reduce(j):
            acc = hist_v[0, pl.ds(j * 16, 16)]
            for l in range(1, _SC_L):
                acc = acc + hist_v[l, pl.ds(j * 16, 16)]
            res_v[0, pl.ds(j * 16, 16)] = acc

        pltpu.sync_copy(res_v, sh_hist.at[pl.ds(sid, 1)])
        plsc.subcore_barrier()
        pltpu.sync_copy(sh_hist, hist_v)
        plsc.subcore_barrier()

        @plsc.parallel_loop(0, _NB // 16, unroll=2)
        def _combine(j):
            acc = hist_v[0, pl.ds(j * 16, 16)]
            for l in range(1, _SC_NS):
                acc = acc + hist_v[l, pl.ds(j * 16, 16)]
            scan_v[pl.ds(j * 16, 16)] = acc

    def _boundary(rank):
        def body(j, carry):
            run, bestg, babove = carry
            g = (_NB // 16 - 1) - j
            gsum = jnp.sum(scan_v[pl.ds(g * 16, 16)])
            newrun = run + gsum
            hit = (bestg < 0) & (newrun >= rank)
            bestg = jnp.where(hit, g, bestg)
            babove = jnp.where(hit, run, babove)
            return newrun, bestg, babove

        _, bg, babove = lax.fori_loop(
            0, _NB // 16, body,
            (jnp.int32(0), jnp.int32(-1), jnp.int32(0)))
        vec = scan_v[pl.ds(bg * 16, 16)]
        rc = lax.rev(jnp.cumsum(lax.rev(vec, (0,))), (0,))
        rr = rank - babove
        ii = jnp.max(jnp.where(rc >= rr, iota16, 0))
        rcii = jnp.max(jnp.where(iota16 == ii, rc, 0))
        vii = jnp.max(jnp.where(iota16 == ii, vec, 0))
        b = bg * 16 + ii
        rank2 = rank - (babove + rcii - vii)
        return b, rank2

    def _phase(data_hbm, out_off):
        _hist_pass(data_hbm, 31, 0, 19, _NB - 1)
        b1, rank2 = _boundary(jnp.int32(N_OUT))
        plsc.subcore_barrier()
        _hist_pass(data_hbm, 19, b1, 8, 2047)
        b2, _ = _boundary(rank2)

        @pl.when(sid == 0)
        def _():
            tb_v[...] = jnp.broadcast_to((b1 << 19) | (b2 << 8), (16,))
            pltpu.sync_copy(tb_v.at[pl.ds(0, 8)],
                            out_hbm.at[pl.ds(out_off, 8)])

    @pl.when(core == 0)
    def _():
        _phase(k_hbm, 0)

    @pl.when(core == 1)
    def _():
        _phase(v_hbm, 8)


def _scale_kernel(k_ref, v_ref, tb_ref, par_ref):
    h = pl.program_id(0)
    t_k = lax.bitcast_convert_type(tb_ref[0, 0], jnp.float32)
    t_v = lax.bitcast_convert_type(tb_ref[0, 8], jnp.float32)
    kabs = jnp.abs(k_ref[0, 0])
    vabs = jnp.abs(v_ref[0, 0])
    km = jnp.where(kabs < t_k, kabs, 0.0)
    rows = lax.broadcasted_iota(jnp.int32, (S, 1), 0)
    vmask = (vabs < t_v) & (rows >= SINK_LENGTH)
    vm = jnp.where(vmask, vabs, 0.0)
    kblk = jnp.max(km, axis=0, keepdims=True)
    vblk = jnp.max(vm, axis=0, keepdims=True)

    @pl.when(h == 0)
    def _():
        par_ref[...] = jnp.zeros_like(par_ref)

    par_ref[0:1, :] = jnp.maximum(par_ref[0:1, :], kblk)
    par_ref[1:2, :] = jnp.maximum(par_ref[1:2, :], vblk)

    @pl.when(h == H - 1)
    def _():
        par_ref[0:1, :] = jnp.maximum(par_ref[0:1, :], 1e-6) / QMAX
        par_ref[1:2, :] = jnp.maximum(par_ref[1:2, :], 1e-6) / QMAX
        par_ref[2:3, :] = jnp.full((1, D), t_k)
        par_ref[3:4, :] = jnp.full((1, D), t_v)


def _params_block(k, v, tb):
    return pl.pallas_call(
        _scale_kernel,
        grid=(H,),
        in_specs=[
            pl.BlockSpec((1, 1, S, D), lambda h: (0, h, 0, 0)),
            pl.BlockSpec((1, 1, S, D), lambda h: (0, h, 0, 0)),
            pl.BlockSpec((1, 16), lambda h: (0, 0)),
        ],
        out_specs=pl.BlockSpec((8, D), lambda h: (0, 0)),
        out_shape=jax.ShapeDtypeStruct((8, D), jnp.float32),
    )(k, v, tb)


def _flash_kernel(params_ref, q_ref, k_ref, v_ref, o_ref, krec_ref, vrec_ref):
    qb = pl.program_id(1)

    @pl.when(qb == 0)
    def _():
        kraw = k_ref[0, 0]
        vraw = v_ref[0, 0]
        ks = params_ref[0:1, :]
        vs = params_ref[1:2, :]
        t_k = params_ref[2:3, :]
        t_v = params_ref[3:4, :]
        kdq = jnp.clip(jnp.round(kraw / ks), -QMAX, QMAX) * ks
        krec_ref[...] = jnp.where(jnp.abs(kraw) >= t_k, kraw, kdq)
        vdq = jnp.clip(jnp.round(vraw / vs), -QMAX, QMAX) * vs
        rows = lax.broadcasted_iota(jnp.int32, (S, 1), 0)
        keep = (jnp.abs(vraw) >= t_v) | (rows < SINK_LENGTH)
        vrec_ref[...] = jnp.where(keep, vraw, vdq)

    qblk = q_ref[0, 0] * (1.0 / math.sqrt(float(D)))
    n_chunks = 4
    half = S // n_chunks
    os_, ms_, ls_ = [], [], []
    for c in range(n_chunks):
        s = lax.dot_general(
            qblk, krec_ref[pl.ds(c * half, half), :], (((1,), (1,)), ((), ())),
            preferred_element_type=jnp.float32,
            precision=lax.Precision.DEFAULT,
        )
        m = jnp.max(s, axis=-1, keepdims=True)
        p = jnp.exp(s - m)
        l = jnp.sum(p, axis=-1, keepdims=True)
        o = lax.dot_general(
            p, vrec_ref[pl.ds(c * half, half), :], (((1,), (0,)), ((), ())),
            preferred_element_type=jnp.float32,
            precision=lax.Precision.DEFAULT,
        )
        os_.append(o); ms_.append(m); ls_.append(l)
    mm = functools.reduce(jnp.maximum, ms_)
    onum = None
    oden = None
    for c in range(n_chunks):
        a = jnp.exp(ms_[c] - mm)
        onum = os_[c] * a if onum is None else onum + os_[c] * a
        oden = ls_[c] * a if oden is None else oden + ls_[c] * a
    o_ref[0, 0] = onum / oden


def _attention(params, q, k, v):
    return pl.pallas_call(
        _flash_kernel,
        grid=(H, S // BLK_Q),
        in_specs=[
            pl.BlockSpec((8, D), lambda h, qb: (0, 0)),
            pl.BlockSpec((1, 1, BLK_Q, D), lambda h, qb: (0, h, qb, 0)),
            pl.BlockSpec((1, 1, S, D), lambda h, qb: (0, h, 0, 0)),
            pl.BlockSpec((1, 1, S, D), lambda h, qb: (0, h, 0, 0)),
        ],
        out_specs=pl.BlockSpec((1, 1, BLK_Q, D), lambda h, qb: (0, h, qb, 0)),
        out_shape=jax.ShapeDtypeStruct((B, H, S, D), jnp.float32),
        scratch_shapes=[
            pltpu.VMEM((S, D), jnp.float32),
            pltpu.VMEM((S, D), jnp.float32),
        ],
    )(params, q, k, v)


def kernel(q_tensor, k_tensor, v_tensor):
    kf = lax.bitcast_convert_type(k_tensor.reshape(-1), jnp.int32)
    vf = lax.bitcast_convert_type(v_tensor.reshape(-1), jnp.int32)
    tb = _sc_thresh(kf, vf).reshape(1, 16)
    params = _params_block(k_tensor, v_tensor, tb)
    return _attention(params, q_tensor, k_tensor, v_tensor)
